# baseline (speedup 1.0000x reference)
import jax
import jax.numpy as jnp
from jax.experimental import pallas as pl
from jax.experimental.pallas import tpu as pltpu

E = 5
H = 384
OUT = 256
B = 128
P = 10
T = 32
F = 15
PT = P * T
RF = P * F
TP = 16
NB = B // TP


def _ln(x, g, b, eps=1e-5):
    m = x.mean(axis=-1, keepdims=True)
    v = ((x - m) ** 2).mean(axis=-1, keepdims=True)
    return (x - m) / jnp.sqrt(v + eps) * g + b


def _body(rf_ref, u_ref, rW1, rb1, rg1, rB1, rW2, rb2, rg2, rB2, rW3, rb3,
          x_ref, eW1_ref, eb1_ref, eW2_ref, eb2_ref, oW1_ref, ob1_ref,
          og_ref, oB_ref, oW2_ref, ob2_ref,
          out_ref, rw_ref, idx_ref, rw_s, pooled_s):
    i = pl.program_id(0)

    @pl.when(i == 0)
    def _router():
        hp = jax.lax.Precision.DEFAULT
        h = jnp.dot(rf_ref[...], rW1[...], precision=hp,
                    preferred_element_type=jnp.float32) + rb1[...]
        h = jax.nn.relu(_ln(h, rg1[...], rB1[...]))
        h = jnp.dot(h, rW2[...], precision=hp,
                    preferred_element_type=jnp.float32) + rb2[...]
        h = jax.nn.relu(_ln(h, rg2[...], rB2[...]))
        logits = jnp.dot(h, rW3[...], precision=hp,
                         preferred_element_type=jnp.float32) + rb3[...]
        scores = logits - jnp.log(-jnp.log(u_ref[...]))
        m = scores[:, 0:1]
        bi = jnp.zeros((B, 1), jnp.int32)
        for e in range(1, E):
            se = scores[:, e:e + 1]
            upd = se > m
            m = jnp.where(upd, se, m)
            bi = jnp.where(upd, e, bi)
        idx_ref[...] = bi
        iota = jax.lax.broadcasted_iota(jnp.int32, (B, E), 1)
        rw = (iota == bi).astype(jnp.float32)
        rw_ref[...] = rw
        rw_s[...] = rw.reshape(NB, TP, E)

    @pl.when((i >= 1) & (i <= NB))
    def _dispatch():
        j = i - 1
        rwb = rw_s[j]
        ones_row = jnp.ones((1, PT), jnp.float32)
        for t in range(TP):
            w1 = rwb[t:t + 1, 0:1] * eW1_ref[0]
            b1 = rwb[t:t + 1, 0:1] * eb1_ref[0]
            for e in range(1, E):
                w1 = w1 + rwb[t:t + 1, e:e + 1] * eW1_ref[e]
                b1 = b1 + rwb[t:t + 1, e:e + 1] * eb1_ref[e]
            xb = x_ref[t].reshape(PT, F)
            h = jnp.dot(xb, w1, preferred_element_type=jnp.float32) + b1
            h = jax.nn.relu(h)
            s = jnp.dot(ones_row, h, preferred_element_type=jnp.float32)
            pooled_s[j, t, :] = (s * (1.0 / PT)).reshape(H)

    @pl.when(i == NB + 1)
    def _combine():
        pooled = pooled_s[...].reshape(B, H)
        rw = rw_s[...].reshape(B, E)
        z = jnp.dot(rw, eb2_ref[...], preferred_element_type=jnp.float32)
        for e in range(E):
            z = z + jnp.dot(pooled * rw[:, e:e + 1], eW2_ref[e],
                            preferred_element_type=jnp.float32)
        p1 = jnp.dot(z, oW1_ref[...], preferred_element_type=jnp.float32) + ob1_ref[...]
        p1 = jax.nn.relu(_ln(p1, og_ref[...], oB_ref[...]))
        out_ref[...] = jnp.dot(p1, oW2_ref[...],
                               preferred_element_type=jnp.float32) + ob2_ref[...]


def kernel(x, gumbel_u, rW1, rb1, rg1, rB1, rW2, rb2, rg2, rB2, rW3, rb3,
           eW1, eb1, eW2, eb2, oW1, ob1, og, oB, oW2, ob2):
    rf = x[:, :, 0, :].reshape(B, RF)
    eb1_r = eb1.reshape(E, 1, H)

    def const(*shape):
        zeros = (0,) * len(shape)
        return pl.BlockSpec(shape, lambda i, z=zeros: z)

    outputs, rw, bi = pl.pallas_call(
        _body,
        grid=(NB + 2,),
        in_specs=[
            const(B, RF), const(B, E),
            const(RF, H), const(H,), const(H,), const(H,),
            const(H, H // 2), const(H // 2,), const(H // 2,), const(H // 2,),
            const(H // 2, E), const(E,),
            pl.BlockSpec((TP, P, T, F),
                         lambda i: (jnp.clip(i - 1, 0, NB - 1), 0, 0, 0)),
            const(E, F, H), const(E, 1, H), const(E, H, H), const(E, H),
            const(H, H // 2), const(H // 2,), const(H // 2,), const(H // 2,),
            const(H // 2, OUT), const(OUT,),
        ],
        out_specs=[const(B, OUT), const(B, E), const(B, 1)],
        out_shape=[
            jax.ShapeDtypeStruct((B, OUT), jnp.float32),
            jax.ShapeDtypeStruct((B, E), jnp.float32),
            jax.ShapeDtypeStruct((B, 1), jnp.int32),
        ],
        scratch_shapes=[
            pltpu.VMEM((NB, TP, E), jnp.float32),
            pltpu.VMEM((NB, TP, H), jnp.float32),
        ],
    )(rf, gumbel_u, rW1, rb1, rg1, rB1, rW2, rb2, rg2, rB2, rW3, rb3,
      x, eW1, eb1_r, eW2, eb2, oW1, ob1, og, oB, oW2, ob2)

    return (outputs, rw, bi.reshape(B))


# speedup vs baseline: 1.5090x; 1.5090x over previous
import jax
import jax.numpy as jnp
from jax.experimental import pallas as pl
from jax.experimental.pallas import tpu as pltpu

E = 5
H = 384
OUT = 256
B = 128
P = 10
T = 32
F = 15
PT = P * T
RF = P * F
TP = 32
NB = B // TP


def _ln(x, g, b, eps=1e-5):
    m = x.mean(axis=-1, keepdims=True)
    v = ((x - m) ** 2).mean(axis=-1, keepdims=True)
    return (x - m) / jnp.sqrt(v + eps) * g + b


def _body(rf_ref, u_ref, rW1, rb1, rg1, rB1, rW2, rb2, rg2, rB2, rW3, rb3,
          x_ref, eW1_ref, eb1_ref, eW2_ref, eb2_ref, oW1_ref, ob1_ref,
          og_ref, oB_ref, oW2_ref, ob2_ref,
          out_ref, rw_ref, idx_ref, rw_s, pooled_s):
    i = pl.program_id(0)

    @pl.when(i == 0)
    def _router():
        hp = jax.lax.Precision.DEFAULT
        h = jnp.dot(rf_ref[...], rW1[...], precision=hp,
                    preferred_element_type=jnp.float32) + rb1[...]
        h = jax.nn.relu(_ln(h, rg1[...], rB1[...]))
        h = jnp.dot(h, rW2[...], precision=hp,
                    preferred_element_type=jnp.float32) + rb2[...]
        h = jax.nn.relu(_ln(h, rg2[...], rB2[...]))
        logits = jnp.dot(h, rW3[...], precision=hp,
                         preferred_element_type=jnp.float32) + rb3[...]
        scores = logits - jnp.log(-jnp.log(u_ref[...]))
        m = scores[:, 0:1]
        bi = jnp.zeros((B, 1), jnp.int32)
        for e in range(1, E):
            se = scores[:, e:e + 1]
            upd = se > m
            m = jnp.where(upd, se, m)
            bi = jnp.where(upd, e, bi)
        idx_ref[...] = bi
        iota = jax.lax.broadcasted_iota(jnp.int32, (B, E), 1)
        rw = (iota == bi).astype(jnp.float32)
        rw_ref[...] = rw
        rw_s[...] = rw.reshape(NB, TP, E)

    @pl.when((i >= 1) & (i <= NB))
    def _dispatch():
        j = i - 1
        rwb = rw_s[j]
        for t in range(TP):
            w1 = rwb[t:t + 1, 0:1] * eW1_ref[0]
            b1 = rwb[t:t + 1, 0:1] * eb1_ref[0]
            for e in range(1, E):
                w1 = w1 + rwb[t:t + 1, e:e + 1] * eW1_ref[e]
                b1 = b1 + rwb[t:t + 1, e:e + 1] * eb1_ref[e]
            xb = x_ref[t].reshape(PT, F)
            h = jnp.dot(xb, w1, preferred_element_type=jnp.float32) + b1
            h = jax.nn.relu(h)
            pooled_s[j, t, :] = jnp.mean(h, axis=0)

    @pl.when(i == NB + 1)
    def _combine():
        pooled = pooled_s[...].reshape(B, H)
        rw = rw_s[...].reshape(B, E)
        z = jnp.dot(rw, eb2_ref[...], preferred_element_type=jnp.float32)
        for e in range(E):
            z = z + jnp.dot(pooled * rw[:, e:e + 1], eW2_ref[e],
                            preferred_element_type=jnp.float32)
        p1 = jnp.dot(z, oW1_ref[...], preferred_element_type=jnp.float32) + ob1_ref[...]
        p1 = jax.nn.relu(_ln(p1, og_ref[...], oB_ref[...]))
        out_ref[...] = jnp.dot(p1, oW2_ref[...],
                               preferred_element_type=jnp.float32) + ob2_ref[...]


def kernel(x, gumbel_u, rW1, rb1, rg1, rB1, rW2, rb2, rg2, rB2, rW3, rb3,
           eW1, eb1, eW2, eb2, oW1, ob1, og, oB, oW2, ob2):
    rf = x[:, :, 0, :].reshape(B, RF)
    eb1_r = eb1.reshape(E, 1, H)

    def const(*shape):
        zeros = (0,) * len(shape)
        return pl.BlockSpec(shape, lambda i, z=zeros: z)

    outputs, rw, bi = pl.pallas_call(
        _body,
        grid=(NB + 2,),
        in_specs=[
            const(B, RF), const(B, E),
            const(RF, H), const(H,), const(H,), const(H,),
            const(H, H // 2), const(H // 2,), const(H // 2,), const(H // 2,),
            const(H // 2, E), const(E,),
            pl.BlockSpec((TP, P, T, F),
                         lambda i: (jnp.clip(i - 1, 0, NB - 1), 0, 0, 0)),
            const(E, F, H), const(E, 1, H), const(E, H, H), const(E, H),
            const(H, H // 2), const(H // 2,), const(H // 2,), const(H // 2,),
            const(H // 2, OUT), const(OUT,),
        ],
        out_specs=[const(B, OUT), const(B, E), const(B, 1)],
        out_shape=[
            jax.ShapeDtypeStruct((B, OUT), jnp.float32),
            jax.ShapeDtypeStruct((B, E), jnp.float32),
            jax.ShapeDtypeStruct((B, 1), jnp.int32),
        ],
        scratch_shapes=[
            pltpu.VMEM((NB, TP, E), jnp.float32),
            pltpu.VMEM((NB, TP, H), jnp.float32),
        ],
    )(rf, gumbel_u, rW1, rb1, rg1, rB1, rW2, rb2, rg2, rB2, rW3, rb3,
      x, eW1, eb1_r, eW2, eb2, oW1, ob1, og, oB, oW2, ob2)

    return (outputs, rw, bi.reshape(B))
